# baseline (speedup 1.0000x reference)
import functools

import jax
import jax.numpy as jnp
from jax import lax
from jax.experimental import pallas as pl
from jax.experimental.pallas import tpu as pltpu
from jax.experimental.pallas import tpu_sc as plsc

N_NODES = 50000
KNN_NUM = 8
EF_DIM = 128
N_GATHER = N_NODES * KNN_NUM

NC, NS = 2, 16
NW = 25
CH = 128
NSLAB = 5
NCH = 25
ROWS_SLAB = NW * NCH * CH
PTS_SLAB = ROWS_SLAB // KNN_NUM

PRE_BLK = 2000
TAIL_ROWS = 3200
TAIL_PTS = TAIL_ROWS // KNN_NUM
TAIL_BLKS = ROWS_SLAB // TAIL_ROWS


def _tail_body(g_ref, xyz_ref, wf_ref, wx_ref, b1_ref, w2_ref, b2_ref, o_ref):
    pre = (
        jnp.dot(g_ref[...], wf_ref[...], preferred_element_type=jnp.float32)
        + lax.dot_general(
            xyz_ref[...],
            wx_ref[...],
            (((0,), (0,)), ((), ())),
            preferred_element_type=jnp.float32,
        )
        + b1_ref[...]
    )
    act = jnp.where(pre >= 0, pre, 0.01 * pre)
    h2 = (
        jnp.dot(act, w2_ref[...], preferred_element_type=jnp.float32)
        + b2_ref[...]
    )
    o_ref[...] = jnp.max(h2.reshape(TAIL_PTS, KNN_NUM, EF_DIM), axis=1)


NBUF = 5
GWIN = 3


def _sc_gather_body(h_hbm, idx_hbm, out_hbm, idx_v, *scr):
    rows = scr[0:NBUF]
    gsem = scr[NBUF : 2 * NBUF]
    ssem = scr[2 * NBUF : 3 * NBUF]
    wid = lax.axis_index("c") * NS + lax.axis_index("s")

    @pl.when(wid < NW)
    def _():
        base_ch = wid * NCH
        pltpu.sync_copy(idx_hbm.at[wid], idx_v)

        def out_slice(c):
            return out_hbm.at[pl.ds((base_ch + c) * CH, CH)]

        def fire_gather(c, b):
            pltpu.async_copy(h_hbm.at[idx_v.at[c]], rows[b], gsem[b])

        def wait_gather(c, b):
            pltpu.make_async_copy(h_hbm.at[idx_v.at[c]], rows[b], gsem[b]).wait()

        def fire_store(c, b):
            pltpu.async_copy(rows[b], out_slice(c), ssem[b])

        def wait_store(c, b):
            pltpu.make_async_copy(rows[b], out_slice(c), ssem[b]).wait()

        def visit(c, b, do_store_wait, do_gather_fire):
            wait_gather(c, b)
            fire_store(c, b)
            b2 = (b + GWIN) % NBUF
            if do_store_wait:
                wait_store(c + GWIN - NBUF, b2)
            if do_gather_fire:
                fire_gather(c + GWIN, b2)

        for c in range(GWIN):
            fire_gather(c, c)
        for b in range(NBUF):
            visit(b, b, do_store_wait=(b + GWIN - NBUF >= 0), do_gather_fire=True)

        def round_body(j, carry):
            for b in range(NBUF):
                visit(NBUF * j + b, b, do_store_wait=True, do_gather_fire=True)
            return carry

        lax.fori_loop(1, NCH // NBUF - 1, round_body, 0)

        for b in range(NBUF):
            c = NCH - NBUF + b
            visit(c, b, do_store_wait=True, do_gather_fire=(c + GWIN < NCH))
        for c in range(NCH + GWIN - NBUF, NCH):
            wait_store(c, c % NBUF)


@functools.lru_cache(maxsize=1)
def _sc_gather():
    return pl.kernel(
        _sc_gather_body,
        out_type=jax.ShapeDtypeStruct((ROWS_SLAB, EF_DIM), jnp.float32),
        mesh=plsc.VectorSubcoreMesh(
            core_axis_name="c", subcore_axis_name="s", num_cores=NC, num_subcores=NS
        ),
        scratch_types=(
            [pltpu.VMEM((NCH, CH), jnp.int32)]
            + [pltpu.VMEM((CH, EF_DIM), jnp.float32) for _ in range(NBUF)]
            + [pltpu.SemaphoreType.DMA for _ in range(2 * NBUF)]
        ),
    )


def kernel(input, KNN_idx, KNN_xyz, W1, b1, W2, b2):
    idx = KNN_idx.astype(jnp.int32)
    w1f_t = W1[:, :EF_DIM].T
    w1x_t = W1[:, EF_DIM:].T
    w2_t = W2.T

    idx_s = idx.reshape(NSLAB, NW, NCH, CH)
    xyz_t = KNN_xyz.T
    gs = [_sc_gather()(input, idx_s[s]) for s in range(NSLAB)]

    b1r = b1.reshape(1, EF_DIM)
    b2r = b2.reshape(1, EF_DIM)
    outs = []
    for s in range(NSLAB):
        xyz_slab = lax.slice_in_dim(xyz_t, s * ROWS_SLAB, (s + 1) * ROWS_SLAB, axis=1)
        outs.append(
            pl.pallas_call(
                _tail_body,
                grid=(TAIL_BLKS,),
                in_specs=[
                    pl.BlockSpec((TAIL_ROWS, EF_DIM), lambda i: (i, 0)),
                    pl.BlockSpec((3, TAIL_ROWS), lambda i: (0, i)),
                    pl.BlockSpec((EF_DIM, EF_DIM), lambda i: (0, 0)),
                    pl.BlockSpec((3, EF_DIM), lambda i: (0, 0)),
                    pl.BlockSpec((1, EF_DIM), lambda i: (0, 0)),
                    pl.BlockSpec((EF_DIM, EF_DIM), lambda i: (0, 0)),
                    pl.BlockSpec((1, EF_DIM), lambda i: (0, 0)),
                ],
                out_specs=pl.BlockSpec((TAIL_PTS, EF_DIM), lambda i: (i, 0)),
                out_shape=jax.ShapeDtypeStruct((PTS_SLAB, EF_DIM), jnp.float32),
            )(gs[s], xyz_slab, w1f_t, w1x_t, b1r, w2_t, b2r)
        )
    return jnp.concatenate(outs)


# speedup vs baseline: 1.1256x; 1.1256x over previous
import functools

import jax
import jax.numpy as jnp
from jax import lax
from jax.experimental import pallas as pl
from jax.experimental.pallas import tpu as pltpu
from jax.experimental.pallas import tpu_sc as plsc

N_NODES = 50000
KNN_NUM = 8
EF_DIM = 128
N_GATHER = N_NODES * KNN_NUM

NC, NS = 2, 16
NW = 25
CH = 128
NSLAB = 5
NCH = 25
ROWS_SLAB = NW * NCH * CH
PTS_SLAB = ROWS_SLAB // KNN_NUM

PRE_BLK = 2000
TAIL_ROWS = 16000
TAIL_PTS = TAIL_ROWS // KNN_NUM
TAIL_BLKS = ROWS_SLAB // TAIL_ROWS


def _tail_body(g_ref, xyz_ref, wf_ref, wx_ref, b1_ref, w2_ref, b2_ref, o_ref):
    pre = (
        jnp.dot(g_ref[...], wf_ref[...], preferred_element_type=jnp.float32)
        + lax.dot_general(
            xyz_ref[...],
            wx_ref[...],
            (((0,), (0,)), ((), ())),
            preferred_element_type=jnp.float32,
        )
        + b1_ref[...]
    )
    act = jnp.where(pre >= 0, pre, 0.01 * pre)
    h2 = (
        jnp.dot(act, w2_ref[...], preferred_element_type=jnp.float32)
        + b2_ref[...]
    )
    o_ref[...] = jnp.max(h2.reshape(TAIL_PTS, KNN_NUM, EF_DIM), axis=1)


NBUF = 5
GWIN = 3


def _sc_gather_body(h_hbm, idx_hbm, out_hbm, idx_v, *scr):
    rows = scr[0:NBUF]
    gsem = scr[NBUF : 2 * NBUF]
    ssem = scr[2 * NBUF : 3 * NBUF]
    wid = lax.axis_index("c") * NS + lax.axis_index("s")

    @pl.when(wid < NW)
    def _():
        base_ch = wid * NCH
        pltpu.sync_copy(idx_hbm.at[wid], idx_v)

        def out_slice(c):
            return out_hbm.at[pl.ds((base_ch + c) * CH, CH)]

        def fire_gather(c, b):
            pltpu.async_copy(h_hbm.at[idx_v.at[c]], rows[b], gsem[b])

        def wait_gather(c, b):
            pltpu.make_async_copy(h_hbm.at[idx_v.at[c]], rows[b], gsem[b]).wait()

        def fire_store(c, b):
            pltpu.async_copy(rows[b], out_slice(c), ssem[b])

        def wait_store(c, b):
            pltpu.make_async_copy(rows[b], out_slice(c), ssem[b]).wait()

        def visit(c, b, do_store_wait, do_gather_fire):
            wait_gather(c, b)
            fire_store(c, b)
            b2 = (b + GWIN) % NBUF
            if do_store_wait:
                wait_store(c + GWIN - NBUF, b2)
            if do_gather_fire:
                fire_gather(c + GWIN, b2)

        for c in range(GWIN):
            fire_gather(c, c)
        for b in range(NBUF):
            visit(b, b, do_store_wait=(b + GWIN - NBUF >= 0), do_gather_fire=True)

        def round_body(j, carry):
            for b in range(NBUF):
                visit(NBUF * j + b, b, do_store_wait=True, do_gather_fire=True)
            return carry

        lax.fori_loop(1, NCH // NBUF - 1, round_body, 0)

        for b in range(NBUF):
            c = NCH - NBUF + b
            visit(c, b, do_store_wait=True, do_gather_fire=(c + GWIN < NCH))
        for c in range(NCH + GWIN - NBUF, NCH):
            wait_store(c, c % NBUF)


@functools.lru_cache(maxsize=1)
def _sc_gather():
    return pl.kernel(
        _sc_gather_body,
        out_type=jax.ShapeDtypeStruct((ROWS_SLAB, EF_DIM), jnp.float32),
        mesh=plsc.VectorSubcoreMesh(
            core_axis_name="c", subcore_axis_name="s", num_cores=NC, num_subcores=NS
        ),
        scratch_types=(
            [pltpu.VMEM((NCH, CH), jnp.int32)]
            + [pltpu.VMEM((CH, EF_DIM), jnp.float32) for _ in range(NBUF)]
            + [pltpu.SemaphoreType.DMA for _ in range(2 * NBUF)]
        ),
    )


def kernel(input, KNN_idx, KNN_xyz, W1, b1, W2, b2):
    idx = KNN_idx.astype(jnp.int32)
    w1f_t = W1[:, :EF_DIM].T
    w1x_t = W1[:, EF_DIM:].T
    w2_t = W2.T

    idx_s = idx.reshape(NSLAB, NW, NCH, CH)
    xyz_t = KNN_xyz.T
    gs = [_sc_gather()(input, idx_s[s]) for s in range(NSLAB)]

    b1r = b1.reshape(1, EF_DIM)
    b2r = b2.reshape(1, EF_DIM)
    outs = []
    for s in range(NSLAB):
        xyz_slab = lax.slice_in_dim(xyz_t, s * ROWS_SLAB, (s + 1) * ROWS_SLAB, axis=1)
        outs.append(
            pl.pallas_call(
                _tail_body,
                grid=(TAIL_BLKS,),
                in_specs=[
                    pl.BlockSpec((TAIL_ROWS, EF_DIM), lambda i: (i, 0)),
                    pl.BlockSpec((3, TAIL_ROWS), lambda i: (0, i)),
                    pl.BlockSpec((EF_DIM, EF_DIM), lambda i: (0, 0)),
                    pl.BlockSpec((3, EF_DIM), lambda i: (0, 0)),
                    pl.BlockSpec((1, EF_DIM), lambda i: (0, 0)),
                    pl.BlockSpec((EF_DIM, EF_DIM), lambda i: (0, 0)),
                    pl.BlockSpec((1, EF_DIM), lambda i: (0, 0)),
                ],
                out_specs=pl.BlockSpec((TAIL_PTS, EF_DIM), lambda i: (i, 0)),
                out_shape=jax.ShapeDtypeStruct((PTS_SLAB, EF_DIM), jnp.float32),
            )(gs[s], xyz_slab, w1f_t, w1x_t, b1r, w2_t, b2r)
        )
    return jnp.concatenate(outs)


# speedup vs baseline: 1.1262x; 1.0005x over previous
import functools

import jax
import jax.numpy as jnp
from jax import lax
from jax.experimental import pallas as pl
from jax.experimental.pallas import tpu as pltpu
from jax.experimental.pallas import tpu_sc as plsc

N_NODES = 50000
KNN_NUM = 8
EF_DIM = 128
N_GATHER = N_NODES * KNN_NUM

NC, NS = 2, 16
NW = 25
CH = 128
NSLAB = 5
NCH = 25
ROWS_SLAB = NW * NCH * CH
PTS_SLAB = ROWS_SLAB // KNN_NUM

TAIL_ROWS = 16000
TAIL_PTS = TAIL_ROWS // KNN_NUM
TAIL_BLKS = ROWS_SLAB // TAIL_ROWS


def _tail_body(g_ref, xyz_ref, wf_ref, wx_ref, b1_ref, w2_ref, b2_ref, o_ref):
    pre = (
        jnp.dot(g_ref[...], wf_ref[...], preferred_element_type=jnp.float32)
        + lax.dot_general(
            xyz_ref[...],
            wx_ref[...],
            (((0,), (0,)), ((), ())),
            preferred_element_type=jnp.float32,
        )
        + b1_ref[...]
    )
    act = jnp.where(pre >= 0, pre, 0.01 * pre)
    h2 = (
        jnp.dot(act, w2_ref[...], preferred_element_type=jnp.float32)
        + b2_ref[...]
    )
    o_ref[...] = jnp.max(h2.reshape(TAIL_PTS, KNN_NUM, EF_DIM), axis=1)


NBUF = 5
GWIN = 3


def _sc_gather_body(h_hbm, idx_hbm, out_hbm, idx_v, *scr):
    rows = scr[0:NBUF]
    gsem = scr[NBUF : 2 * NBUF]
    ssem = scr[2 * NBUF : 3 * NBUF]
    wid = lax.axis_index("c") * NS + lax.axis_index("s")

    @pl.when(wid < NW)
    def _():
        base_ch = wid * NCH
        pltpu.sync_copy(idx_hbm.at[wid], idx_v)

        def out_slice(c):
            return out_hbm.at[pl.ds((base_ch + c) * CH, CH)]

        def fire_gather(c, b):
            pltpu.async_copy(h_hbm.at[idx_v.at[c]], rows[b], gsem[b])

        def wait_gather(c, b):
            pltpu.make_async_copy(h_hbm.at[idx_v.at[c]], rows[b], gsem[b]).wait()

        def fire_store(c, b):
            pltpu.async_copy(rows[b], out_slice(c), ssem[b])

        def wait_store(c, b):
            pltpu.make_async_copy(rows[b], out_slice(c), ssem[b]).wait()

        def visit(c, b, do_store_wait, do_gather_fire):
            wait_gather(c, b)
            fire_store(c, b)
            b2 = (b + GWIN) % NBUF
            if do_store_wait:
                wait_store(c + GWIN - NBUF, b2)
            if do_gather_fire:
                fire_gather(c + GWIN, b2)

        for c in range(GWIN):
            fire_gather(c, c)
        for b in range(NBUF):
            visit(b, b, do_store_wait=(b + GWIN - NBUF >= 0), do_gather_fire=True)

        def round_body(j, carry):
            for b in range(NBUF):
                visit(NBUF * j + b, b, do_store_wait=True, do_gather_fire=True)
            return carry

        lax.fori_loop(1, NCH // NBUF - 1, round_body, 0)

        for b in range(NBUF):
            c = NCH - NBUF + b
            visit(c, b, do_store_wait=True, do_gather_fire=(c + GWIN < NCH))
        for c in range(NCH + GWIN - NBUF, NCH):
            wait_store(c, c % NBUF)


@functools.lru_cache(maxsize=1)
def _sc_gather():
    return pl.kernel(
        _sc_gather_body,
        out_type=jax.ShapeDtypeStruct((ROWS_SLAB, EF_DIM), jnp.float32),
        mesh=plsc.VectorSubcoreMesh(
            core_axis_name="c", subcore_axis_name="s", num_cores=NC, num_subcores=NS
        ),
        scratch_types=(
            [pltpu.VMEM((NCH, CH), jnp.int32)]
            + [pltpu.VMEM((CH, EF_DIM), jnp.float32) for _ in range(NBUF)]
            + [pltpu.SemaphoreType.DMA for _ in range(2 * NBUF)]
        ),
    )


def kernel(input, KNN_idx, KNN_xyz, W1, b1, W2, b2):
    idx = KNN_idx.astype(jnp.int32)
    w1f_t = W1[:, :EF_DIM].T
    w1x_t = W1[:, EF_DIM:].T
    w2_t = W2.T

    idx_s = idx.reshape(NSLAB, NW, NCH, CH)
    xyz_t = KNN_xyz.T
    gs = [_sc_gather()(input, idx_s[s]) for s in range(NSLAB)]

    b1r = b1.reshape(1, EF_DIM)
    b2r = b2.reshape(1, EF_DIM)
    outs = []
    for s in range(NSLAB):
        xyz_slab = lax.slice_in_dim(xyz_t, s * ROWS_SLAB, (s + 1) * ROWS_SLAB, axis=1)
        outs.append(
            pl.pallas_call(
                _tail_body,
                grid=(TAIL_BLKS,),
                in_specs=[
                    pl.BlockSpec((TAIL_ROWS, EF_DIM), lambda i: (i, 0)),
                    pl.BlockSpec((3, TAIL_ROWS), lambda i: (0, i)),
                    pl.BlockSpec((EF_DIM, EF_DIM), lambda i: (0, 0)),
                    pl.BlockSpec((3, EF_DIM), lambda i: (0, 0)),
                    pl.BlockSpec((1, EF_DIM), lambda i: (0, 0)),
                    pl.BlockSpec((EF_DIM, EF_DIM), lambda i: (0, 0)),
                    pl.BlockSpec((1, EF_DIM), lambda i: (0, 0)),
                ],
                out_specs=pl.BlockSpec((TAIL_PTS, EF_DIM), lambda i: (i, 0)),
                out_shape=jax.ShapeDtypeStruct((PTS_SLAB, EF_DIM), jnp.float32),
            )(gs[s], xyz_slab, w1f_t, w1x_t, b1r, w2_t, b2r)
        )
    return jnp.concatenate(outs)
